# baseline (speedup 1.0000x reference)
import jax
import jax.numpy as jnp
from jax.experimental import pallas as pl
from jax.experimental.pallas import tpu as pltpu

_NBG = 2
_N = 4096
_FT = 256
_HID = 64
_BM = 256
_NM = _N // _BM

_TB = (((1,), (1,)), ((), ()))


def _fused_body(feat_ref, shuf_ref, w_ref, adj_ref, ht_ref, wd_ref, bd_ref,
                wa_ref, ba_ref, s1_ref, s2_ref,
                out0_ref, out1_ref, out2_ref, reg_ref,
                s_scr, hall_scr):
    g = pl.program_id(0)
    m = pl.program_id(1)

    @pl.when(m == 0)
    def _compute_s():
        w = w_ref[0]
        s_scr[:, :_HID] = jnp.dot(feat_ref[0, 0], w,
                                  preferred_element_type=jnp.float32)
        s_scr[:, _HID:] = jnp.dot(shuf_ref[0, 0], w,
                                  preferred_element_type=jnp.float32)

    acc = jnp.dot(adj_ref[0, 0], s_scr[:],
                  preferred_element_type=jnp.float32)
    hall_scr[pl.ds(g * _N + m * _BM, _BM), :] = jnp.maximum(acc, 0.0)

    @pl.when((g == _NBG - 1) & (m == _NM - 1))
    def _epilogue():
        h1_0 = hall_scr[pl.ds(0, _N), :_HID]
        h2_0 = hall_scr[pl.ds(0, _N), _HID:]
        h1_1 = hall_scr[pl.ds(_N, _N), :_HID]
        h2_1 = hall_scr[pl.ds(_N, _N), _HID:]
        s1 = s1_ref[:]
        s2 = s2_ref[:]

        inv_n = jnp.float32(1.0 / _N)
        cm0 = jnp.sum(h1_0, axis=0, keepdims=True) * inv_n
        cm1 = jnp.sum(h1_1, axis=0, keepdims=True) * inv_n
        c0 = jax.nn.sigmoid(cm0)
        c1 = jax.nn.sigmoid(cm1)
        ca = jax.nn.sigmoid((cm0 + cm1) * 0.5)

        wd = wd_ref[:]
        wa = wa_ref[:]
        v0 = jax.lax.dot_general(c0, wd, _TB,
                                 preferred_element_type=jnp.float32)
        v1 = jax.lax.dot_general(c1, wd, _TB,
                                 preferred_element_type=jnp.float32)
        va = jax.lax.dot_general(ca, wa, _TB,
                                 preferred_element_type=jnp.float32)
        b0 = bd_ref[:]
        ba = ba_ref[:]

        out0_ref[:, :_N] = jax.lax.dot_general(
            v0, h1_0, _TB, preferred_element_type=jnp.float32) + b0 + s1
        out0_ref[:, _N:] = jax.lax.dot_general(
            v0, h2_0, _TB, preferred_element_type=jnp.float32) + b0 + s2
        out1_ref[:, :_N] = jax.lax.dot_general(
            v1, h1_1, _TB, preferred_element_type=jnp.float32) + b0 + s1
        out1_ref[:, _N:] = jax.lax.dot_general(
            v1, h2_1, _TB, preferred_element_type=jnp.float32) + b0 + s2

        h1a = (h1_0 + h1_1) * 0.5
        h2a = (h2_0 + h2_1) * 0.5
        out2_ref[:, :_N] = jax.lax.dot_general(
            va, h1a, _TB, preferred_element_type=jnp.float32) + ba + s1
        out2_ref[:, _N:] = jax.lax.dot_general(
            va, h2a, _TB, preferred_element_type=jnp.float32) + ba + s2

        h0 = ht_ref[0]
        d1 = h0 - h1a
        d2 = h0 - h2a
        reg_ref[:, :] = (jnp.sum(d1 * d1, keepdims=True)
                         - jnp.sum(d2 * d2, keepdims=True))


def kernel(feature, adj, shuf, sparse, msk, samp_bias1, samp_bias2,
           W_gcn, W_disc, b_disc, W_discAll, b_discAll, H):
    f32 = jnp.float32

    out0, out1, out2, reg = pl.pallas_call(
        _fused_body,
        grid=(_NBG, _NM),
        in_specs=[
            pl.BlockSpec((1, 1, _N, _FT), lambda g, m: (g, 0, 0, 0)),
            pl.BlockSpec((1, 1, _N, _FT), lambda g, m: (g, 0, 0, 0)),
            pl.BlockSpec((1, _FT, _HID), lambda g, m: (g, 0, 0)),
            pl.BlockSpec((1, 1, _BM, _N), lambda g, m: (g, 0, m, 0)),
            pl.BlockSpec((1, _N, _HID), lambda g, m: (0, 0, 0)),
            pl.BlockSpec((_HID, _HID), lambda g, m: (0, 0)),
            pl.BlockSpec((1, 1), lambda g, m: (0, 0)),
            pl.BlockSpec((_HID, _HID), lambda g, m: (0, 0)),
            pl.BlockSpec((1, 1), lambda g, m: (0, 0)),
            pl.BlockSpec((1, _N), lambda g, m: (0, 0)),
            pl.BlockSpec((1, _N), lambda g, m: (0, 0)),
        ],
        out_specs=(
            pl.BlockSpec((1, 2 * _N), lambda g, m: (0, 0)),
            pl.BlockSpec((1, 2 * _N), lambda g, m: (0, 0)),
            pl.BlockSpec((1, 2 * _N), lambda g, m: (0, 0)),
            pl.BlockSpec((1, 1), lambda g, m: (0, 0)),
        ),
        out_shape=(
            jax.ShapeDtypeStruct((1, 2 * _N), f32),
            jax.ShapeDtypeStruct((1, 2 * _N), f32),
            jax.ShapeDtypeStruct((1, 2 * _N), f32),
            jax.ShapeDtypeStruct((1, 1), f32),
        ),
        scratch_shapes=[
            pltpu.VMEM((_N, 2 * _HID), f32),
            pltpu.VMEM((_NBG * _N, 2 * _HID), f32),
        ],
        compiler_params=pltpu.CompilerParams(
            dimension_semantics=("arbitrary", "arbitrary"),
            vmem_limit_bytes=100 * 1024 * 1024),
    )(feature, shuf, W_gcn, adj, H, W_disc, b_disc.reshape(1, 1),
      W_discAll, b_discAll.reshape(1, 1), samp_bias1, samp_bias2)

    return out0, out1, out2, reg.reshape(())


# speedup vs baseline: 1.1636x; 1.1636x over previous
import jax
import jax.numpy as jnp
from jax.experimental import pallas as pl
from jax.experimental.pallas import tpu as pltpu

_NBG = 2
_N = 4096
_FT = 256
_HID = 64
_BM = 512
_NM = _N // _BM

_TB = (((1,), (1,)), ((), ()))


def _fused_body(feat_ref, shuf_ref, w_ref, adj_ref, ht_ref, wd_ref, bd_ref,
                wa_ref, ba_ref, s1_ref, s2_ref,
                out0_ref, out1_ref, out2_ref, reg_ref,
                s_scr, hall_scr, cm_scr):
    g = pl.program_id(0)
    m = pl.program_id(1)

    @pl.when(m == 0)
    def _compute_s():
        w = w_ref[0]
        s_scr[:, :_HID] = jnp.dot(feat_ref[0, 0], w,
                                  preferred_element_type=jnp.float32)
        s_scr[:, _HID:] = jnp.dot(shuf_ref[0, 0], w,
                                  preferred_element_type=jnp.float32)

    acc = jnp.dot(adj_ref[0, 0], s_scr[:],
                  preferred_element_type=jnp.float32)
    hall_scr[pl.ds(g * _N + m * _BM, _BM), :] = jnp.maximum(acc, 0.0)

    inv_n = jnp.float32(1.0 / _N)
    wd = wd_ref[:]
    b0 = bd_ref[:]
    s1 = s1_ref[:]
    s2 = s2_ref[:]

    @pl.when((g == 0) & (m == _NM - 1))
    def _logits_g0():
        h1_0 = hall_scr[pl.ds(0, _N), :_HID]
        h2_0 = hall_scr[pl.ds(0, _N), _HID:]
        cm0 = jnp.sum(h1_0, axis=0, keepdims=True) * inv_n
        cm_scr[:, :] = cm0
        c0 = jax.nn.sigmoid(cm0)
        v0 = jax.lax.dot_general(c0, wd, _TB,
                                 preferred_element_type=jnp.float32)
        out0_ref[:, :_N] = jax.lax.dot_general(
            v0, h1_0, _TB, preferred_element_type=jnp.float32) + b0 + s1
        out0_ref[:, _N:] = jax.lax.dot_general(
            v0, h2_0, _TB, preferred_element_type=jnp.float32) + b0 + s2

    @pl.when((g == _NBG - 1) & (m == _NM - 1))
    def _epilogue():
        h1_0 = hall_scr[pl.ds(0, _N), :_HID]
        h2_0 = hall_scr[pl.ds(0, _N), _HID:]
        h1_1 = hall_scr[pl.ds(_N, _N), :_HID]
        h2_1 = hall_scr[pl.ds(_N, _N), _HID:]

        cm0 = cm_scr[:, :]
        cm1 = jnp.sum(h1_1, axis=0, keepdims=True) * inv_n
        c1 = jax.nn.sigmoid(cm1)
        ca = jax.nn.sigmoid((cm0 + cm1) * 0.5)

        wa = wa_ref[:]
        v1 = jax.lax.dot_general(c1, wd, _TB,
                                 preferred_element_type=jnp.float32)
        va = jax.lax.dot_general(ca, wa, _TB,
                                 preferred_element_type=jnp.float32)
        ba = ba_ref[:]

        out1_ref[:, :_N] = jax.lax.dot_general(
            v1, h1_1, _TB, preferred_element_type=jnp.float32) + b0 + s1
        out1_ref[:, _N:] = jax.lax.dot_general(
            v1, h2_1, _TB, preferred_element_type=jnp.float32) + b0 + s2

        h1a = (h1_0 + h1_1) * 0.5
        h2a = (h2_0 + h2_1) * 0.5
        out2_ref[:, :_N] = jax.lax.dot_general(
            va, h1a, _TB, preferred_element_type=jnp.float32) + ba + s1
        out2_ref[:, _N:] = jax.lax.dot_general(
            va, h2a, _TB, preferred_element_type=jnp.float32) + ba + s2

        h0 = ht_ref[0]
        d1 = h0 - h1a
        d2 = h0 - h2a
        reg_ref[:, :] = (jnp.sum(d1 * d1, keepdims=True)
                         - jnp.sum(d2 * d2, keepdims=True))


def kernel(feature, adj, shuf, sparse, msk, samp_bias1, samp_bias2,
           W_gcn, W_disc, b_disc, W_discAll, b_discAll, H):
    f32 = jnp.float32

    out0, out1, out2, reg = pl.pallas_call(
        _fused_body,
        grid=(_NBG, _NM),
        in_specs=[
            pl.BlockSpec((1, 1, _N, _FT), lambda g, m: (g, 0, 0, 0)),
            pl.BlockSpec((1, 1, _N, _FT), lambda g, m: (g, 0, 0, 0)),
            pl.BlockSpec((1, _FT, _HID), lambda g, m: (g, 0, 0)),
            pl.BlockSpec((1, 1, _BM, _N), lambda g, m: (g, 0, m, 0)),
            pl.BlockSpec((1, _N, _HID), lambda g, m: (0, 0, 0)),
            pl.BlockSpec((_HID, _HID), lambda g, m: (0, 0)),
            pl.BlockSpec((1, 1), lambda g, m: (0, 0)),
            pl.BlockSpec((_HID, _HID), lambda g, m: (0, 0)),
            pl.BlockSpec((1, 1), lambda g, m: (0, 0)),
            pl.BlockSpec((1, _N), lambda g, m: (0, 0)),
            pl.BlockSpec((1, _N), lambda g, m: (0, 0)),
        ],
        out_specs=(
            pl.BlockSpec((1, 2 * _N), lambda g, m: (0, 0)),
            pl.BlockSpec((1, 2 * _N), lambda g, m: (0, 0)),
            pl.BlockSpec((1, 2 * _N), lambda g, m: (0, 0)),
            pl.BlockSpec((1, 1), lambda g, m: (0, 0)),
        ),
        out_shape=(
            jax.ShapeDtypeStruct((1, 2 * _N), f32),
            jax.ShapeDtypeStruct((1, 2 * _N), f32),
            jax.ShapeDtypeStruct((1, 2 * _N), f32),
            jax.ShapeDtypeStruct((1, 1), f32),
        ),
        scratch_shapes=[
            pltpu.VMEM((_N, 2 * _HID), f32),
            pltpu.VMEM((_NBG * _N, 2 * _HID), f32),
            pltpu.VMEM((1, _HID), f32),
        ],
        compiler_params=pltpu.CompilerParams(
            dimension_semantics=("arbitrary", "arbitrary"),
            vmem_limit_bytes=100 * 1024 * 1024),
    )(feature, shuf, W_gcn, adj, H, W_disc, b_disc.reshape(1, 1),
      W_discAll, b_discAll.reshape(1, 1), samp_bias1, samp_bias2)

    return out0, out1, out2, reg.reshape(())


# speedup vs baseline: 1.1730x; 1.0081x over previous
import jax
import jax.numpy as jnp
from jax.experimental import pallas as pl
from jax.experimental.pallas import tpu as pltpu

_NBG = 2
_N = 4096
_FT = 256
_HID = 64
_BM = 512
_NM = _N // _BM

_TB = (((1,), (1,)), ((), ()))


def _fused_body(feat_ref, shuf_ref, w_ref, adj_ref, ht_ref, wd_ref, bd_ref,
                wa_ref, ba_ref, s1_ref, s2_ref,
                out0_ref, out1_ref, out2_ref, reg_ref,
                s_scr, hall_scr, cm_scr):
    g = pl.program_id(0)
    m = pl.program_id(1)

    @pl.when(m == 0)
    def _compute_s():
        w = w_ref[0]
        s_scr[:, :_HID] = jnp.dot(feat_ref[0, 0], w,
                                  preferred_element_type=jnp.float32)
        s_scr[:, _HID:] = jnp.dot(shuf_ref[0, 0], w,
                                  preferred_element_type=jnp.float32)

    acc = jnp.dot(adj_ref[0, 0], s_scr[:],
                  preferred_element_type=jnp.float32)
    relu_blk = jnp.maximum(acc, 0.0)
    hall_scr[pl.ds(g * _N + m * _BM, _BM), :] = relu_blk

    inv_n = jnp.float32(1.0 / _N)
    wd = wd_ref[:]
    b0 = bd_ref[:]
    s1 = s1_ref[:]
    s2 = s2_ref[:]

    @pl.when(g == _NBG - 1)
    def _accumulate():
        h1_1b = relu_blk[:, :_HID]
        h2_1b = relu_blk[:, _HID:]
        h1_0b = hall_scr[pl.ds(m * _BM, _BM), :_HID]
        h2_0b = hall_scr[pl.ds(m * _BM, _BM), _HID:]
        htb = ht_ref[0, pl.ds(m * _BM, _BM), :]
        e1 = htb - (h1_0b + h1_1b) * 0.5
        e2 = htb - (h2_0b + h2_1b) * 0.5
        part = jnp.concatenate(
            [jnp.sum(h1_1b, axis=0, keepdims=True),
             jnp.sum(e1 * e1, axis=0, keepdims=True),
             jnp.sum(e2 * e2, axis=0, keepdims=True)], axis=0)

        @pl.when(m == 0)
        def _init():
            cm_scr[1:4, :] = part

        @pl.when(m != 0)
        def _add():
            cm_scr[1:4, :] = cm_scr[1:4, :] + part

    @pl.when((g == 0) & (m == _NM - 1))
    def _logits_g0():
        h1_0 = hall_scr[pl.ds(0, _N), :_HID]
        h2_0 = hall_scr[pl.ds(0, _N), _HID:]
        cm0 = jnp.sum(h1_0, axis=0, keepdims=True) * inv_n
        cm_scr[0:1, :] = cm0
        c0 = jax.nn.sigmoid(cm0)
        v0 = jax.lax.dot_general(c0, wd, _TB,
                                 preferred_element_type=jnp.float32)
        out0_ref[:, :_N] = jax.lax.dot_general(
            v0, h1_0, _TB, preferred_element_type=jnp.float32) + b0 + s1
        out0_ref[:, _N:] = jax.lax.dot_general(
            v0, h2_0, _TB, preferred_element_type=jnp.float32) + b0 + s2

    @pl.when((g == _NBG - 1) & (m == _NM - 1))
    def _epilogue():
        h1_0 = hall_scr[pl.ds(0, _N), :_HID]
        h2_0 = hall_scr[pl.ds(0, _N), _HID:]
        h1_1 = hall_scr[pl.ds(_N, _N), :_HID]
        h2_1 = hall_scr[pl.ds(_N, _N), _HID:]

        cm0 = cm_scr[0:1, :]
        cm1 = cm_scr[1:2, :] * inv_n
        c1 = jax.nn.sigmoid(cm1)
        ca = jax.nn.sigmoid((cm0 + cm1) * 0.5)

        wa = wa_ref[:]
        v1 = jax.lax.dot_general(c1, wd, _TB,
                                 preferred_element_type=jnp.float32)
        va = jax.lax.dot_general(ca, wa, _TB,
                                 preferred_element_type=jnp.float32)
        ba = ba_ref[:]

        out1_ref[:, :_N] = jax.lax.dot_general(
            v1, h1_1, _TB, preferred_element_type=jnp.float32) + b0 + s1
        out1_ref[:, _N:] = jax.lax.dot_general(
            v1, h2_1, _TB, preferred_element_type=jnp.float32) + b0 + s2

        h1a = (h1_0 + h1_1) * 0.5
        h2a = (h2_0 + h2_1) * 0.5
        out2_ref[:, :_N] = jax.lax.dot_general(
            va, h1a, _TB, preferred_element_type=jnp.float32) + ba + s1
        out2_ref[:, _N:] = jax.lax.dot_general(
            va, h2a, _TB, preferred_element_type=jnp.float32) + ba + s2

        reg_ref[:, :] = (jnp.sum(cm_scr[2:3, :], keepdims=True)
                         - jnp.sum(cm_scr[3:4, :], keepdims=True))


def kernel(feature, adj, shuf, sparse, msk, samp_bias1, samp_bias2,
           W_gcn, W_disc, b_disc, W_discAll, b_discAll, H):
    f32 = jnp.float32

    out0, out1, out2, reg = pl.pallas_call(
        _fused_body,
        grid=(_NBG, _NM),
        in_specs=[
            pl.BlockSpec((1, 1, _N, _FT), lambda g, m: (g, 0, 0, 0)),
            pl.BlockSpec((1, 1, _N, _FT), lambda g, m: (g, 0, 0, 0)),
            pl.BlockSpec((1, _FT, _HID), lambda g, m: (g, 0, 0)),
            pl.BlockSpec((1, 1, _BM, _N), lambda g, m: (g, 0, m, 0)),
            pl.BlockSpec((1, _N, _HID), lambda g, m: (0, 0, 0)),
            pl.BlockSpec((_HID, _HID), lambda g, m: (0, 0)),
            pl.BlockSpec((1, 1), lambda g, m: (0, 0)),
            pl.BlockSpec((_HID, _HID), lambda g, m: (0, 0)),
            pl.BlockSpec((1, 1), lambda g, m: (0, 0)),
            pl.BlockSpec((1, _N), lambda g, m: (0, 0)),
            pl.BlockSpec((1, _N), lambda g, m: (0, 0)),
        ],
        out_specs=(
            pl.BlockSpec((1, 2 * _N), lambda g, m: (0, 0)),
            pl.BlockSpec((1, 2 * _N), lambda g, m: (0, 0)),
            pl.BlockSpec((1, 2 * _N), lambda g, m: (0, 0)),
            pl.BlockSpec((1, 1), lambda g, m: (0, 0)),
        ),
        out_shape=(
            jax.ShapeDtypeStruct((1, 2 * _N), f32),
            jax.ShapeDtypeStruct((1, 2 * _N), f32),
            jax.ShapeDtypeStruct((1, 2 * _N), f32),
            jax.ShapeDtypeStruct((1, 1), f32),
        ),
        scratch_shapes=[
            pltpu.VMEM((_N, 2 * _HID), f32),
            pltpu.VMEM((_NBG * _N, 2 * _HID), f32),
            pltpu.VMEM((4, _HID), f32),
        ],
        compiler_params=pltpu.CompilerParams(
            dimension_semantics=("arbitrary", "arbitrary"),
            vmem_limit_bytes=100 * 1024 * 1024),
    )(feature, shuf, W_gcn, adj, H, W_disc, b_disc.reshape(1, 1),
      W_discAll, b_discAll.reshape(1, 1), samp_bias1, samp_bias2)

    return out0, out1, out2, reg.reshape(())
